# baseline (speedup 1.0000x reference)
import jax
import jax.numpy as jnp
from jax.experimental import pallas as pl

N_GRAPHS = 128
NODES_PER = 96
EDGES_PER = 256
D = 128
MSG_HID = 256
SDIM = 64
PROP_STEPS = 3
TEMP = 0.1
SINK_ITERS = 20

PAIRS = N_GRAPHS // 2
PN = 2 * NODES_PER
PE = 2 * EDGES_PER
PP = 8
STEPS = PAIRS // PP


def _body(nf_ref, ef_ref, lfc_ref, lfr_ref, ltc_ref, ltr_ref,
          Wne_ref, bne_ref, Wee_ref, bee_ref,
          Wm1a_ref, Wm1b_ref, Wm1c_ref, bm1_ref, Wm2_ref, bm2_ref,
          Wn1a_ref, Wn1b_ref, Wn1c_ref, bn1_ref, Wn2_ref, bn2_ref,
          Ws1_ref, bs1_ref, Ws2_ref, bs2_ref, out_ref):
    relu = lambda x: jnp.maximum(x, 0.0)

    Wm1a = Wm1a_ref[...]
    Wm1b = Wm1b_ref[...]
    Wm2 = Wm2_ref[...]
    bm2 = bm2_ref[...]

    col_iota = jax.lax.broadcasted_iota(jnp.int32, (PE, PN), 1)
    row_iota = jax.lax.broadcasted_iota(jnp.int32, (PN, PE), 0)

    las, qs, cs = [], [], []
    for pp in range(PP):
        nf = nf_ref[0, pp]
        ef = ef_ref[0, pp]
        lfc = lfc_ref[0, pp]
        lfr = lfr_ref[0, pp]
        ltc = ltc_ref[0, pp]
        ltr = ltr_ref[0, pp]

        h = relu(nf @ Wne_ref[...] + bne_ref[...])
        e = relu(ef @ Wee_ref[...] + bee_ref[...])
        e1c = e @ Wm1c_ref[...] + bm1_ref[...]

        Gf = (lfc == col_iota).astype(jnp.float32)
        Gt = (ltc == col_iota).astype(jnp.float32)
        GfT = (lfr == row_iota).astype(jnp.float32)
        GtT = (ltr == row_iota).astype(jnp.float32)

        for _ in range(PROP_STEPS):
            src = Gf @ h
            dst = Gt @ h
            sA = src @ Wm1a
            sB = src @ Wm1b
            dA = dst @ Wm1a
            dB = dst @ Wm1b
            m_fwd = relu(sA + dB + e1c) @ Wm2 + bm2
            m_rev = relu(dA + sB + e1c) @ Wm2 + bm2
            agg_f = GtT @ m_fwd
            agg_r = GfT @ m_rev
            u = h @ Wn1a_ref[...] + agg_f @ Wn1b_ref[...] \
                + agg_r @ Wn1c_ref[...] + bn1_ref[...]
            h = relu(u) @ Wn2_ref[...] + bn2_ref[...]

        src = Gf @ h
        dst = Gt @ h
        ee = relu(src @ Wm1a + dst @ Wm1b + e1c) @ Wm2 + bm2
        q = ee[0:EDGES_PER]
        c = ee[EDGES_PER:PE]

        mq = relu(q @ Ws1_ref[...] + bs1_ref[...]) @ Ws2_ref[...] \
            + bs2_ref[...]
        mc = relu(c @ Ws1_ref[...] + bs1_ref[...]) @ Ws2_ref[...] \
            + bs2_ref[...]
        la = jax.lax.dot_general(mq, mc, (((1,), (1,)), ((), ()))) \
            * (1.0 / TEMP)
        las.append(la)
        qs.append(q)
        cs.append(c)

    def sink_step(las):
        out = []
        for la in las:
            m1 = jnp.max(la, axis=1, keepdims=True)
            la = la - (m1 + jnp.log(
                jnp.sum(jnp.exp(la - m1), axis=1, keepdims=True)))
            m0 = jnp.max(la, axis=0, keepdims=True)
            la = la - (m0 + jnp.log(
                jnp.sum(jnp.exp(la - m0), axis=0, keepdims=True)))
            out.append(la)
        return tuple(out)

    las = tuple(las)
    for _ in range(SINK_ITERS):
        las = sink_step(las)

    for pp in range(PP):
        transport = jnp.exp(las[pp])
        aligned = transport @ cs[pp]
        score = -jnp.sum(relu(qs[pp] - aligned), keepdims=True)
        out_ref[0, pp] = score


def kernel(node_features, edge_features, from_idx, to_idx,
           W_ne, b_ne, W_ee, b_ee, Wm1, bm1, Wm2, bm2,
           Wn1, bn1, Wn2, bn2, Ws1, bs1, Ws2, bs2):
    E = from_idx.shape[0]
    nf = node_features.reshape(STEPS, PP, PN, D)
    ef = edge_features.reshape(STEPS, PP, PE, D)
    pair = (jnp.arange(E, dtype=jnp.int32) // PE) * PN
    lf = from_idx.astype(jnp.int32) - pair
    lt = to_idx.astype(jnp.int32) - pair
    lfc = lf.reshape(STEPS, PP, PE, 1)
    lfr = lf.reshape(STEPS, PP, 1, PE)
    ltc = lt.reshape(STEPS, PP, PE, 1)
    ltr = lt.reshape(STEPS, PP, 1, PE)

    row = lambda b: b.reshape(1, -1)
    full = lambda a: pl.BlockSpec(a.shape, lambda p: (0,) * a.ndim)
    data = lambda a: pl.BlockSpec((1,) + a.shape[1:],
                                  lambda p: (p, 0, 0, 0))

    operands = [
        nf, ef, lfc, lfr, ltc, ltr,
        W_ne, row(b_ne), W_ee, row(b_ee),
        Wm1[0:D], Wm1[D:2 * D], Wm1[2 * D:3 * D], row(bm1), Wm2, row(bm2),
        Wn1[0:D], Wn1[D:2 * D], Wn1[2 * D:3 * D], row(bn1), Wn2, row(bn2),
        Ws1, row(bs1), Ws2, row(bs2),
    ]
    in_specs = [data(a) for a in operands[:6]] + [full(a) for a in operands[6:]]

    out = pl.pallas_call(
        _body,
        grid=(STEPS,),
        in_specs=in_specs,
        out_specs=pl.BlockSpec((1, PP, 1, 1), lambda p: (p, 0, 0, 0)),
        out_shape=jax.ShapeDtypeStruct((STEPS, PP, 1, 1), jnp.float32),
    )(*operands)
    return out.reshape(PAIRS)


# speedup vs baseline: 1.0842x; 1.0842x over previous
import jax
import jax.numpy as jnp
from jax.experimental import pallas as pl

N_GRAPHS = 128
NODES_PER = 96
EDGES_PER = 256
D = 128
MSG_HID = 256
SDIM = 64
PROP_STEPS = 3
TEMP = 0.1
SINK_ITERS = 20

PAIRS = N_GRAPHS // 2
PN = 2 * NODES_PER
PE = 2 * EDGES_PER
PP = 4
STEPS = PAIRS // PP


def _body(nf_ref, ef_ref, lfc_ref, lfr_ref, ltc_ref, ltr_ref,
          Wne_ref, bne_ref, Wee_ref, bee_ref,
          Wm1a_ref, Wm1b_ref, Wm1c_ref, bm1_ref, Wm2_ref, bm2_ref,
          Wn1a_ref, Wn1b_ref, Wn1c_ref, bn1_ref, Wn2_ref, bn2_ref,
          Ws1_ref, bs1_ref, Ws2_ref, bs2_ref, out_ref):
    relu = lambda x: jnp.maximum(x, 0.0)

    Wm1a = Wm1a_ref[...]
    Wm1b = Wm1b_ref[...]
    Wm2 = Wm2_ref[...]
    bm2 = bm2_ref[...]

    col_iota = jax.lax.broadcasted_iota(jnp.int32, (PE, PN), 1)
    row_iota = jax.lax.broadcasted_iota(jnp.int32, (PN, PE), 0)

    las, qs, cs = [], [], []
    for pp in range(PP):
        nf = nf_ref[0, pp]
        ef = ef_ref[0, pp]
        lfc = lfc_ref[0, pp]
        lfr = lfr_ref[0, pp]
        ltc = ltc_ref[0, pp]
        ltr = ltr_ref[0, pp]

        h = relu(nf @ Wne_ref[...] + bne_ref[...])
        e = relu(ef @ Wee_ref[...] + bee_ref[...])
        e1c = e @ Wm1c_ref[...] + bm1_ref[...]

        Gf = (lfc == col_iota).astype(jnp.float32)
        Gt = (ltc == col_iota).astype(jnp.float32)
        GfT = (lfr == row_iota).astype(jnp.float32)
        GtT = (ltr == row_iota).astype(jnp.float32)

        for _ in range(PROP_STEPS):
            src = Gf @ h
            dst = Gt @ h
            sA = src @ Wm1a
            sB = src @ Wm1b
            dA = dst @ Wm1a
            dB = dst @ Wm1b
            m_fwd = relu(sA + dB + e1c) @ Wm2 + bm2
            m_rev = relu(dA + sB + e1c) @ Wm2 + bm2
            agg_f = GtT @ m_fwd
            agg_r = GfT @ m_rev
            u = h @ Wn1a_ref[...] + agg_f @ Wn1b_ref[...] \
                + agg_r @ Wn1c_ref[...] + bn1_ref[...]
            h = relu(u) @ Wn2_ref[...] + bn2_ref[...]

        src = Gf @ h
        dst = Gt @ h
        ee = relu(src @ Wm1a + dst @ Wm1b + e1c) @ Wm2 + bm2
        q = ee[0:EDGES_PER]
        c = ee[EDGES_PER:PE]

        mq = relu(q @ Ws1_ref[...] + bs1_ref[...]) @ Ws2_ref[...] \
            + bs2_ref[...]
        mc = relu(c @ Ws1_ref[...] + bs1_ref[...]) @ Ws2_ref[...] \
            + bs2_ref[...]
        la = jax.lax.dot_general(mq, mc, (((1,), (1,)), ((), ()))) \
            * (1.0 / TEMP)
        las.append(la)
        qs.append(q)
        cs.append(c)

    def sink_step(las):
        out = []
        for la in las:
            m1 = jnp.max(la, axis=1, keepdims=True)
            la = la - (m1 + jnp.log(
                jnp.sum(jnp.exp(la - m1), axis=1, keepdims=True)))
            m0 = jnp.max(la, axis=0, keepdims=True)
            la = la - (m0 + jnp.log(
                jnp.sum(jnp.exp(la - m0), axis=0, keepdims=True)))
            out.append(la)
        return tuple(out)

    las = tuple(las)
    for _ in range(SINK_ITERS):
        las = sink_step(las)

    for pp in range(PP):
        transport = jnp.exp(las[pp])
        aligned = transport @ cs[pp]
        score = -jnp.sum(relu(qs[pp] - aligned), keepdims=True)
        out_ref[0, pp] = score


def kernel(node_features, edge_features, from_idx, to_idx,
           W_ne, b_ne, W_ee, b_ee, Wm1, bm1, Wm2, bm2,
           Wn1, bn1, Wn2, bn2, Ws1, bs1, Ws2, bs2):
    E = from_idx.shape[0]
    nf = node_features.reshape(STEPS, PP, PN, D)
    ef = edge_features.reshape(STEPS, PP, PE, D)
    pair = (jnp.arange(E, dtype=jnp.int32) // PE) * PN
    lf = from_idx.astype(jnp.int32) - pair
    lt = to_idx.astype(jnp.int32) - pair
    lfc = lf.reshape(STEPS, PP, PE, 1)
    lfr = lf.reshape(STEPS, PP, 1, PE)
    ltc = lt.reshape(STEPS, PP, PE, 1)
    ltr = lt.reshape(STEPS, PP, 1, PE)

    row = lambda b: b.reshape(1, -1)
    full = lambda a: pl.BlockSpec(a.shape, lambda p: (0,) * a.ndim)
    data = lambda a: pl.BlockSpec((1,) + a.shape[1:],
                                  lambda p: (p, 0, 0, 0))

    operands = [
        nf, ef, lfc, lfr, ltc, ltr,
        W_ne, row(b_ne), W_ee, row(b_ee),
        Wm1[0:D], Wm1[D:2 * D], Wm1[2 * D:3 * D], row(bm1), Wm2, row(bm2),
        Wn1[0:D], Wn1[D:2 * D], Wn1[2 * D:3 * D], row(bn1), Wn2, row(bn2),
        Ws1, row(bs1), Ws2, row(bs2),
    ]
    in_specs = [data(a) for a in operands[:6]] + [full(a) for a in operands[6:]]

    out = pl.pallas_call(
        _body,
        grid=(STEPS,),
        in_specs=in_specs,
        out_specs=pl.BlockSpec((1, PP, 1, 1), lambda p: (p, 0, 0, 0)),
        out_shape=jax.ShapeDtypeStruct((STEPS, PP, 1, 1), jnp.float32),
    )(*operands)
    return out.reshape(PAIRS)
